# baseline (speedup 1.0000x reference)
import functools

import jax
import jax.numpy as jnp
from jax import lax
from jax.experimental import pallas as pl
from jax.experimental.pallas import tpu as pltpu

CH = 32


def _body(row_ref, col_ref, out_hbm, buf0, buf1, sem0, sem1, *, B, e, H, W):
    n_dim = 2 * e
    HW = H * W
    bufs = (buf0, buf1)
    sems = (sem0, sem1)
    n_chunks = n_dim // CH
    half = e // CH
    for k in range(n_chunks):
        buf, sem = bufs[k % 2], sems[k % 2]
        if k >= 2:
            pltpu.make_async_copy(
                buf, out_hbm.at[:, pl.ds((k - 2) * CH, CH), :], sem
            ).wait()
        if k < half:
            tblk = col_ref[:W, pl.ds(k * CH, CH)].T
            content = jnp.broadcast_to(tblk[:, None, :], (CH, H, W))
        else:
            tblk = row_ref[:H, pl.ds(k * CH - e, CH)].T
            content = jnp.broadcast_to(tblk[:, :, None], (CH, H, W))
        content = content.reshape(CH, HW)
        buf[...] = jnp.broadcast_to(content[None], (B, CH, HW))
        pltpu.make_async_copy(
            buf, out_hbm.at[:, pl.ds(k * CH, CH), :], sem
        ).start()
    for k in (n_chunks - 2, n_chunks - 1):
        pltpu.make_async_copy(
            bufs[k % 2], out_hbm.at[:, pl.ds(k * CH, CH), :], sems[k % 2]
        ).wait()


def kernel(x, row_embed, col_embed):
    B = x.shape[0]
    H, W = x.shape[-2], x.shape[-1]
    e = row_embed.shape[1]
    n_dim = 2 * e
    out = pl.pallas_call(
        functools.partial(_body, B=B, e=e, H=H, W=W),
        in_specs=[
            pl.BlockSpec(memory_space=pltpu.MemorySpace.VMEM),
            pl.BlockSpec(memory_space=pltpu.MemorySpace.VMEM),
        ],
        out_specs=pl.BlockSpec(memory_space=pltpu.MemorySpace.HBM),
        out_shape=jax.ShapeDtypeStruct((B, n_dim, H * W), row_embed.dtype),
        scratch_shapes=[
            pltpu.VMEM((B, CH, H * W), row_embed.dtype),
            pltpu.VMEM((B, CH, H * W), row_embed.dtype),
            pltpu.SemaphoreType.DMA,
            pltpu.SemaphoreType.DMA,
        ],
    )(row_embed, col_embed)
    return out.reshape(B, n_dim, H, W)


# speedup vs baseline: 1.0349x; 1.0349x over previous
import functools

import jax
import jax.numpy as jnp
from jax.experimental import pallas as pl
from jax.experimental.pallas import tpu as pltpu


def _pos_body(row_ref, col_ref, out_hbm, scratch, sem, *, H, W, B):
    n_dim, HW = scratch.shape
    e = n_dim // 2
    col_t = col_ref[:W, :].T
    row_t = row_ref[:H, :].T
    scratch[:e, :] = jnp.broadcast_to(col_t[:, None, :], (e, H, W)).reshape(e, HW)
    scratch[e:, :] = jnp.broadcast_to(row_t[:, :, None], (e, H, W)).reshape(e, HW)
    for b in range(B):
        pltpu.make_async_copy(scratch, out_hbm.at[b], sem).start()
    for b in range(B):
        pltpu.make_async_copy(scratch, out_hbm.at[b], sem).wait()


def kernel(x, row_embed, col_embed):
    B = x.shape[0]
    H, W = x.shape[-2], x.shape[-1]
    e = row_embed.shape[1]
    n_dim = 2 * e
    out = pl.pallas_call(
        functools.partial(_pos_body, H=H, W=W, B=B),
        in_specs=[
            pl.BlockSpec(memory_space=pltpu.MemorySpace.VMEM),
            pl.BlockSpec(memory_space=pltpu.MemorySpace.VMEM),
        ],
        out_specs=pl.BlockSpec(memory_space=pltpu.MemorySpace.HBM),
        out_shape=jax.ShapeDtypeStruct((B, n_dim, H * W), row_embed.dtype),
        scratch_shapes=[
            pltpu.VMEM((n_dim, H * W), row_embed.dtype),
            pltpu.SemaphoreType.DMA,
        ],
    )(row_embed, col_embed)
    return out.reshape(B, n_dim, H, W)
